# Initial kernel scaffold; baseline (speedup 1.0000x reference)
import jax
import jax.numpy as jnp
from jax.experimental import pallas as pl


def kernel(x):
    raise NotImplementedError("write your pallas kernel here")


# speedup vs baseline: 16.9684x; 16.9684x over previous
import functools

import jax
import jax.numpy as jnp
from jax import lax
from jax.experimental import pallas as pl
from jax.experimental.pallas import tpu as pltpu
from jax.experimental.pallas import tpu_sc as plsc

_RATIO = 0.5
_NC = 2
_NS = 16
_L = 16
_NW = _NC * _NS
_BLK = 8192


def _hist_body(x_hbm, pref_hbm, out_hbm, buf, hist, pref_v, *,
               n_total, b_bins, sh_pref, sh_idx, dual):
    per_tile = n_total // _NW
    n_blk = per_tile // _BLK
    wid = lax.axis_index("s") * _NC + lax.axis_index("c")
    base = wid * per_tile
    nbt = (2 if dual else 1) * _L * b_bins

    zeros16 = jnp.zeros((_L,), jnp.int32)

    def zero_body(i, c):
        hist[pl.ds(i * _L, _L)] = zeros16
        return c

    lax.fori_loop(0, nbt // _L, zero_body, 0)

    pltpu.sync_copy(pref_hbm, pref_v)
    pa = pref_v[pl.ds(0, _L)]
    pb = pref_v[pl.ds(_L, _L)]

    lane_base = lax.iota(jnp.int32, _L) * b_bins
    ones = jnp.ones((_L,), jnp.int32)
    sign_mask = jnp.int32(0x7FFFFFFF)
    bmask = jnp.int32(b_bins - 1)
    off_b = jnp.int32(_L * b_bins)

    def blk_body(i, c):
        pltpu.sync_copy(x_hbm.at[pl.ds(base + i * _BLK, _BLK)], buf)

        def g_body(g, cc):
            v = buf[pl.ds(g * _L, _L)]
            u = lax.bitwise_and(plsc.bitcast(v, jnp.int32), sign_mask)
            idx = lane_base + lax.bitwise_and(
                lax.shift_right_logical(u, sh_idx), bmask)
            if dual:
                p = lax.shift_right_logical(u, sh_pref)
                plsc.addupdate_scatter(hist, [idx], ones, mask=(p == pa))
                plsc.addupdate_scatter(hist, [idx + off_b], ones,
                                       mask=(p == pb))
            else:
                plsc.addupdate_scatter(hist, [idx], ones)
            return cc

        lax.fori_loop(0, _BLK // _L, g_body, 0)
        return c

    lax.fori_loop(0, n_blk, blk_body, 0)
    pltpu.sync_copy(hist, out_hbm.at[wid])


def _make_hist(n_total, b_bins, sh_pref, sh_idx, dual):
    mesh = plsc.VectorSubcoreMesh(core_axis_name="c", subcore_axis_name="s",
                                  num_cores=_NC, num_subcores=_NS)
    nbt = (2 if dual else 1) * _L * b_bins
    body = functools.partial(_hist_body, n_total=n_total, b_bins=b_bins,
                             sh_pref=sh_pref, sh_idx=sh_idx, dual=dual)
    return pl.kernel(
        body,
        out_type=jax.ShapeDtypeStruct((_NW, nbt), jnp.int32),
        mesh=mesh,
        compiler_params=pltpu.CompilerParams(needs_layout_passes=False),
        scratch_types=[
            pltpu.VMEM((_BLK,), jnp.float32),
            pltpu.VMEM((nbt,), jnp.int32),
            pltpu.VMEM((2 * _L,), jnp.int32),
        ],
    )


def _advance(g, rank):
    c = jnp.cumsum(g)
    b = jnp.searchsorted(c, rank, side="right").astype(jnp.int32)
    return b, rank - (c[b] - g[b])


def _mask_kernel(t_ref, x_ref, o_ref):
    xv = x_ref[...]
    o_ref[...] = jnp.where(jnp.abs(xv) > t_ref[0, 0], xv, 0.0)


@jax.jit
def kernel(x):
    shape = x.shape
    n = x.size
    xf = x.reshape(n)

    pos = _RATIO * (n - 1)
    k = int(pos)
    frac = pos - k

    hist1 = _make_hist(n, 4096, 31, 19, False)
    hist2 = _make_hist(n, 2048, 19, 8, True)
    hist3 = _make_hist(n, 256, 8, 0, True)

    zero_pref = jnp.zeros((2 * _L,), jnp.int32)
    h1 = hist1(xf, zero_pref).reshape(_NW, _L, 4096).sum((0, 1))
    ba, ra = _advance(h1, jnp.int32(k))
    bb, rb = _advance(h1, jnp.int32(k + 1))

    pref2 = jnp.concatenate([jnp.full((_L,), ba), jnp.full((_L,), bb)])
    h2 = hist2(xf, pref2).reshape(_NW, 2, _L, 2048).sum((0, 2))
    b2a, ra = _advance(h2[0], ra)
    b2b, rb = _advance(h2[1], rb)

    p3a = (ba << 11) | b2a
    p3b = (bb << 11) | b2b
    pref3 = jnp.concatenate([jnp.full((_L,), p3a), jnp.full((_L,), p3b)])
    h3 = hist3(xf, pref3).reshape(_NW, 2, _L, 256).sum((0, 2))
    b3a, _ = _advance(h3[0], ra)
    b3b, _ = _advance(h3[1], rb)

    va = lax.bitcast_convert_type((p3a << 8) | b3a, jnp.float32)
    vb = lax.bitcast_convert_type((p3b << 8) | b3b, jnp.float32)
    t = va + (vb - va) * jnp.float32(frac)

    rows = n // 2048
    blk_rows = 256
    out = pl.pallas_call(
        _mask_kernel,
        grid=(rows // blk_rows,),
        in_specs=[
            pl.BlockSpec(memory_space=pltpu.SMEM),
            pl.BlockSpec((blk_rows, 2048), lambda i: (i, 0)),
        ],
        out_specs=pl.BlockSpec((blk_rows, 2048), lambda i: (i, 0)),
        out_shape=jax.ShapeDtypeStruct((rows, 2048), jnp.float32),
    )(t.reshape(1, 1), x.reshape(rows, 2048))
    return out.reshape(shape)


# speedup vs baseline: 26.3438x; 1.5525x over previous
import functools

import jax
import jax.numpy as jnp
from jax import lax
from jax.experimental import pallas as pl
from jax.experimental.pallas import tpu as pltpu
from jax.experimental.pallas import tpu_sc as plsc

_RATIO = 0.5
_NC = 2
_NS = 16
_L = 16
_NW = _NC * _NS
_BLK = 8192


def _hist_body(x_hbm, pref_hbm, out_hbm, buf, hist, pref_v, *,
               n_total, b_bins, sh_pref, sh_idx, dual):
    per_tile = n_total // _NW
    n_blk = per_tile // _BLK
    wid = lax.axis_index("s") * _NC + lax.axis_index("c")
    base = wid * per_tile
    nbt = (2 if dual else 1) * b_bins

    zeros16 = jnp.zeros((_L,), jnp.int32)

    def zero_body(i, c):
        for t in range(8):
            hist[pl.ds((i * 8 + t) * _L, _L)] = zeros16
        return c

    lax.fori_loop(0, nbt // (8 * _L), zero_body, 0)

    pltpu.sync_copy(pref_hbm, pref_v)
    pa = pref_v[pl.ds(0, _L)]
    pb = pref_v[pl.ds(_L, _L)]

    ones = jnp.ones((_L,), jnp.int32)
    sign_mask = jnp.int32(0x7FFFFFFF)
    bmask = jnp.int32(b_bins - 1)
    off_b = jnp.int32(b_bins)

    def blk_body(i, c):
        pltpu.sync_copy(x_hbm.at[pl.ds(base + i * _BLK, _BLK)], buf)

        def g_body(g, cc):
            for t in range(4):
                v = buf[pl.ds((g * 4 + t) * _L, _L)]
                u = lax.bitwise_and(plsc.bitcast(v, jnp.int32), sign_mask)
                idx = lax.bitwise_and(lax.shift_right_logical(u, sh_idx),
                                      bmask)
                if dual:
                    p = lax.shift_right_logical(u, sh_pref)
                    plsc.addupdate_scatter(hist, [idx], ones, mask=(p == pa))
                    plsc.addupdate_scatter(hist, [idx + off_b], ones,
                                           mask=(p == pb))
                else:
                    plsc.addupdate_scatter(hist, [idx], ones)
            return cc

        lax.fori_loop(0, _BLK // (4 * _L), g_body, 0)
        return c

    lax.fori_loop(0, n_blk, blk_body, 0)
    pltpu.sync_copy(hist, out_hbm.at[wid])


def _make_hist(n_total, b_bins, sh_pref, sh_idx, dual):
    mesh = plsc.VectorSubcoreMesh(core_axis_name="c", subcore_axis_name="s",
                                  num_cores=_NC, num_subcores=_NS)
    nbt = (2 if dual else 1) * b_bins
    body = functools.partial(_hist_body, n_total=n_total, b_bins=b_bins,
                             sh_pref=sh_pref, sh_idx=sh_idx, dual=dual)
    return pl.kernel(
        body,
        out_type=jax.ShapeDtypeStruct((_NW, nbt), jnp.int32),
        mesh=mesh,
        compiler_params=pltpu.CompilerParams(needs_layout_passes=False),
        scratch_types=[
            pltpu.VMEM((_BLK,), jnp.float32),
            pltpu.VMEM((nbt,), jnp.int32),
            pltpu.VMEM((2 * _L,), jnp.int32),
        ],
    )


def _advance(g, rank):
    c = jnp.cumsum(g)
    b = jnp.searchsorted(c, rank, side="right").astype(jnp.int32)
    return b, rank - (c[b] - g[b])


def _mask_kernel(t_ref, x_ref, o_ref):
    xv = x_ref[...]
    o_ref[...] = jnp.where(jnp.abs(xv) > t_ref[0, 0], xv, 0.0)


@jax.jit
def kernel(x):
    shape = x.shape
    n = x.size
    xf = x.reshape(n)

    pos = _RATIO * (n - 1)
    k = int(pos)
    frac = pos - k

    hist1 = _make_hist(n, 65536, 31, 15, False)
    hist2 = _make_hist(n, 32768, 15, 0, True)

    zero_pref = jnp.zeros((2 * _L,), jnp.int32)
    h1 = hist1(xf, zero_pref).sum(0)
    ba, ra = _advance(h1, jnp.int32(k))
    bb, rb = _advance(h1, jnp.int32(k + 1))

    pref2 = jnp.concatenate([jnp.full((_L,), ba), jnp.full((_L,), bb)])
    h2 = hist2(xf, pref2).reshape(_NW, 2, 32768).sum(0)
    b2a, _ = _advance(h2[0], ra)
    b2b, _ = _advance(h2[1], rb)

    va = lax.bitcast_convert_type((ba << 15) | b2a, jnp.float32)
    vb = lax.bitcast_convert_type((bb << 15) | b2b, jnp.float32)
    t = va + (vb - va) * jnp.float32(frac)

    rows = n // 2048
    blk_rows = 256
    out = pl.pallas_call(
        _mask_kernel,
        grid=(rows // blk_rows,),
        in_specs=[
            pl.BlockSpec(memory_space=pltpu.SMEM),
            pl.BlockSpec((blk_rows, 2048), lambda i: (i, 0)),
        ],
        out_specs=pl.BlockSpec((blk_rows, 2048), lambda i: (i, 0)),
        out_shape=jax.ShapeDtypeStruct((rows, 2048), jnp.float32),
    )(t.reshape(1, 1), x.reshape(rows, 2048))
    return out.reshape(shape)


# speedup vs baseline: 30.3357x; 1.1515x over previous
import functools

import jax
import jax.numpy as jnp
from jax import lax
from jax.experimental import pallas as pl
from jax.experimental.pallas import tpu as pltpu
from jax.experimental.pallas import tpu_sc as plsc

_RATIO = 0.5
_NC = 2
_NS = 16
_L = 16
_NW = _NC * _NS
_BLK = 8192


def _hist_body(x_hbm, pref_hbm, out_hbm, buf0, buf1, hist, pref_v,
               sem0, sem1, *, n_total, b_bins, sh_pref, sh_idx, dual):
    per_tile = n_total // _NW
    n_blk = per_tile // _BLK
    wid = lax.axis_index("s") * _NC + lax.axis_index("c")
    base = wid * per_tile
    nbt = (2 if dual else 1) * b_bins

    def issue(i, buf, sem):
        pltpu.async_copy(x_hbm.at[pl.ds(base + i * _BLK, _BLK)], buf, sem)

    def drain(buf, sem):
        pltpu.make_async_copy(x_hbm.at[pl.ds(base, _BLK)], buf, sem).wait()

    issue(0, buf0, sem0)

    zeros16 = jnp.zeros((_L,), jnp.int32)

    def zero_body(i, c):
        for t in range(8):
            hist[pl.ds((i * 8 + t) * _L, _L)] = zeros16
        return c

    lax.fori_loop(0, nbt // (8 * _L), zero_body, 0)

    pltpu.sync_copy(pref_hbm, pref_v)
    pa = pref_v[pl.ds(0, _L)]
    pb = pref_v[pl.ds(_L, _L)]

    ones = jnp.ones((_L,), jnp.int32)
    sign_mask = jnp.int32(0x7FFFFFFF)
    bmask = jnp.int32(b_bins - 1)
    off_b = jnp.int32(b_bins)

    def consume(buf):
        def g_body(g, cc):
            for t in range(8):
                v = buf[pl.ds((g * 8 + t) * _L, _L)]
                u = lax.bitwise_and(plsc.bitcast(v, jnp.int32), sign_mask)
                idx = lax.bitwise_and(lax.shift_right_logical(u, sh_idx),
                                      bmask)
                if dual:
                    p = lax.shift_right_logical(u, sh_pref)
                    plsc.addupdate_scatter(hist, [idx], ones, mask=(p == pa))
                    plsc.addupdate_scatter(hist, [idx + off_b], ones,
                                           mask=(p == pb))
                else:
                    plsc.addupdate_scatter(hist, [idx], ones)
            return cc

        lax.fori_loop(0, _BLK // (8 * _L), g_body, 0)

    def blk_body(j, c):
        i0 = 2 * j
        issue(i0 + 1, buf1, sem1)
        drain(buf0, sem0)
        consume(buf0)

        @pl.when(i0 + 2 < n_blk)
        def _():
            issue(i0 + 2, buf0, sem0)

        drain(buf1, sem1)
        consume(buf1)
        return c

    lax.fori_loop(0, n_blk // 2, blk_body, 0)
    pltpu.sync_copy(hist, out_hbm.at[wid])


def _make_hist(n_total, b_bins, sh_pref, sh_idx, dual):
    mesh = plsc.VectorSubcoreMesh(core_axis_name="c", subcore_axis_name="s",
                                  num_cores=_NC, num_subcores=_NS)
    nbt = (2 if dual else 1) * b_bins
    body = functools.partial(_hist_body, n_total=n_total, b_bins=b_bins,
                             sh_pref=sh_pref, sh_idx=sh_idx, dual=dual)
    return pl.kernel(
        body,
        out_type=jax.ShapeDtypeStruct((_NW, nbt), jnp.int32),
        mesh=mesh,
        compiler_params=pltpu.CompilerParams(needs_layout_passes=False),
        scratch_types=[
            pltpu.VMEM((_BLK,), jnp.float32),
            pltpu.VMEM((_BLK,), jnp.float32),
            pltpu.VMEM((nbt,), jnp.int32),
            pltpu.VMEM((2 * _L,), jnp.int32),
            pltpu.SemaphoreType.DMA,
            pltpu.SemaphoreType.DMA,
        ],
    )


def _advance(g, rank):
    c = jnp.cumsum(g)
    b = jnp.searchsorted(c, rank, side="right").astype(jnp.int32)
    return b, rank - (c[b] - g[b])


def _mask_kernel(t_ref, x_ref, o_ref):
    xv = x_ref[...]
    o_ref[...] = jnp.where(jnp.abs(xv) > t_ref[0, 0], xv, 0.0)


@jax.jit
def kernel(x):
    shape = x.shape
    n = x.size
    xf = x.reshape(n)

    pos = _RATIO * (n - 1)
    k = int(pos)
    frac = pos - k

    hist1 = _make_hist(n, 65536, 31, 15, False)
    hist2 = _make_hist(n, 32768, 15, 0, True)

    zero_pref = jnp.zeros((2 * _L,), jnp.int32)
    h1 = hist1(xf, zero_pref).sum(0)
    ba, ra = _advance(h1, jnp.int32(k))
    bb, rb = _advance(h1, jnp.int32(k + 1))

    pref2 = jnp.concatenate([jnp.full((_L,), ba), jnp.full((_L,), bb)])
    h2 = hist2(xf, pref2).reshape(_NW, 2, 32768).sum(0)
    b2a, _ = _advance(h2[0], ra)
    b2b, _ = _advance(h2[1], rb)

    va = lax.bitcast_convert_type((ba << 15) | b2a, jnp.float32)
    vb = lax.bitcast_convert_type((bb << 15) | b2b, jnp.float32)
    t = va + (vb - va) * jnp.float32(frac)

    rows = n // 2048
    blk_rows = 256
    out = pl.pallas_call(
        _mask_kernel,
        grid=(rows // blk_rows,),
        in_specs=[
            pl.BlockSpec(memory_space=pltpu.SMEM),
            pl.BlockSpec((blk_rows, 2048), lambda i: (i, 0)),
        ],
        out_specs=pl.BlockSpec((blk_rows, 2048), lambda i: (i, 0)),
        out_shape=jax.ShapeDtypeStruct((rows, 2048), jnp.float32),
    )(t.reshape(1, 1), x.reshape(rows, 2048))
    return out.reshape(shape)
